# baseline (speedup 1.0000x reference)
import jax
import jax.numpy as jnp
import numpy as np
from jax import lax
from jax.experimental import pallas as pl

EPS = 1e-08
_ALLOC0 = float(np.float32(1.0) - np.float32(EPS))
_ALLOC1 = float((np.float32(1.0) - np.float32(EPS)) * np.float32(EPS))


def _write_head_kernel(mem_ref, ctrl_ref, out_ref):
    flat = mem_ref[0]
    ctrl = ctrl_ref[0]
    w = 64

    keys = ctrl[:, 0:w]
    erase = jax.nn.sigmoid(ctrl[:, w:2 * w])
    add = ctrl[:, 2 * w:3 * w]
    beta = jax.nn.softplus(ctrl[:, -3:-2])
    alloc_gate = jax.nn.sigmoid(ctrl[:, -2:-1])
    write_gate = jax.nn.sigmoid(ctrl[:, -1:])

    k2 = jnp.concatenate([keys, keys], axis=-1)
    e2 = jnp.concatenate([erase, erase], axis=-1)
    a2 = jnp.concatenate([add, add], axis=-1)

    si = lax.broadcasted_iota(jnp.int32, (128, 128), 0)
    li = lax.broadcasted_iota(jnp.int32, (128, 128), 1)
    pmask = (si // w) == (li // w)
    pmat = jnp.where(pmask, 1.0, 0.0)
    kcol = jnp.broadcast_to(k2, (128, 128)).T
    kmat = jnp.where(pmask, kcol, 0.0)

    dot = jnp.dot(flat, kmat, precision=lax.Precision.HIGHEST)
    nrm2 = jnp.dot(flat * flat, pmat, precision=lax.Precision.HIGHEST)

    key_norm = jnp.sqrt(jnp.sum(keys * keys))
    scores = (dot * beta) / (key_norm * jnp.sqrt(nrm2) + EPS)

    smax = jnp.max(scores)
    e = jnp.exp(scores - smax)
    content = e * (64.0 / jnp.sum(e))

    wwc = (write_gate * (1.0 - alloc_gate)) * content
    out_ref[0] = flat - wwc * (flat * e2 - a2)

    lane = lax.broadcasted_iota(jnp.int32, (1, 128), 1)
    alloc_row = jnp.where(lane < w, _ALLOC0, _ALLOC1)
    ww0 = wwc[0:1, :] + (write_gate * alloc_gate) * alloc_row
    row0 = flat[0:1, :]
    out_ref[0, 0:1, :] = row0 - ww0 * (row0 * e2 - a2)


def kernel(memory, controls, read_weights):
    b, c, w = memory.shape
    r = (c * w) // 128
    n = controls.shape[-1]
    flat = memory.reshape(b, r, 128)
    ctrl3 = controls.reshape(b, 1, n)
    out = pl.pallas_call(
        _write_head_kernel,
        grid=(b,),
        in_specs=[
            pl.BlockSpec((1, r, 128), lambda i: (i, 0, 0)),
            pl.BlockSpec((1, 1, n), lambda i: (i, 0, 0)),
        ],
        out_specs=pl.BlockSpec((1, r, 128), lambda i: (i, 0, 0)),
        out_shape=jax.ShapeDtypeStruct((b, r, 128), memory.dtype),
    )(flat, ctrl3)
    return out.reshape(b, c, w)


# speedup vs baseline: 5.6891x; 5.6891x over previous
import jax
import jax.numpy as jnp
import numpy as np
from jax import lax
from jax.experimental import pallas as pl

EPS = 1e-08
_LOG_EPS = float(np.log(np.float32(EPS)))


def _write_head_kernel(mem_ref, ctrl_ref, out_ref):
    memt = mem_ref[0]
    ctrl = ctrl_ref[0]
    w, c = memt.shape

    keys = ctrl[:, 0:w]
    erase = jax.nn.sigmoid(ctrl[:, w:2 * w])
    add = ctrl[:, 2 * w:3 * w]
    beta = jax.nn.softplus(ctrl[:, -3:-2])
    alloc_gate = jax.nn.sigmoid(ctrl[:, -2:-1])
    write_gate = jax.nn.sigmoid(ctrl[:, -1:])

    dot = jnp.dot(keys, memt, precision=lax.Precision.HIGHEST)
    ones_row = jnp.ones((1, w), dtype=memt.dtype)
    nrm2 = jnp.dot(ones_row, memt * memt,
                   precision=lax.Precision.HIGHEST)

    key_norm = jnp.sqrt(jnp.sum(keys * keys))
    scores = (dot * beta) / (key_norm * jnp.sqrt(nrm2) + EPS)

    smax = jnp.max(scores)
    e = jnp.exp(scores - smax)
    content_w = e * ((write_gate * (1.0 - alloc_gate)) / jnp.sum(e))

    idx = lax.broadcasted_iota(jnp.int32, (1, c), 1).astype(jnp.float32)
    alloc = (1.0 - EPS) * jnp.exp(idx * _LOG_EPS)
    ww = content_w + (write_gate * alloc_gate) * alloc

    ecol = jnp.broadcast_to(erase, (w, w)).T[:, 0:1]
    acol = jnp.broadcast_to(add, (w, w)).T[:, 0:1]
    out_ref[0] = memt - ww * (memt * ecol - acol)


def kernel(memory, controls, read_weights):
    b, c, w = memory.shape
    n = controls.shape[-1]
    memt = jnp.swapaxes(memory, 1, 2)
    ctrl3 = controls.reshape(b, 1, n)
    out_t = pl.pallas_call(
        _write_head_kernel,
        grid=(b,),
        in_specs=[
            pl.BlockSpec((1, w, c), lambda i: (i, 0, 0)),
            pl.BlockSpec((1, 1, n), lambda i: (i, 0, 0)),
        ],
        out_specs=pl.BlockSpec((1, w, c), lambda i: (i, 0, 0)),
        out_shape=jax.ShapeDtypeStruct((b, w, c), memory.dtype),
    )(memt, ctrl3)
    return jnp.swapaxes(out_t, 1, 2)


# speedup vs baseline: 9.0755x; 1.5952x over previous
import jax
import jax.numpy as jnp
import numpy as np
from jax import lax
from jax.experimental import pallas as pl

EPS = 1e-08
_LOG_EPS = float(np.log(np.float32(EPS)))


def _write_head_kernel(mem_ref, ctrl_ref, out_ref):
    memt = mem_ref[0]
    ctrl = ctrl_ref[0]
    w, c = memt.shape

    keys = ctrl[:, 0:w]
    erase = jax.nn.sigmoid(ctrl[:, w:2 * w])
    add = ctrl[:, 2 * w:3 * w]
    beta = jax.nn.softplus(ctrl[:, -3:-2])
    alloc_gate = jax.nn.sigmoid(ctrl[:, -2:-1])
    write_gate = jax.nn.sigmoid(ctrl[:, -1:])

    dot = jnp.dot(keys, memt)
    ones_row = jnp.ones((1, w), dtype=memt.dtype)
    nrm2 = jnp.dot(ones_row, memt * memt)

    key_norm = jnp.sqrt(jnp.sum(keys * keys))
    scores = (dot * beta) / (key_norm * jnp.sqrt(nrm2) + EPS)

    smax = jnp.max(scores)
    e = jnp.exp(scores - smax)
    content_w = e * ((write_gate * (1.0 - alloc_gate)) / jnp.sum(e))

    idx = lax.broadcasted_iota(jnp.int32, (1, c), 1).astype(jnp.float32)
    alloc = (1.0 - EPS) * jnp.exp(idx * _LOG_EPS)
    ww = content_w + (write_gate * alloc_gate) * alloc

    ecol = jnp.broadcast_to(erase, (w, w)).T[:, 0:1]
    acol = jnp.broadcast_to(add, (w, w)).T[:, 0:1]
    out_ref[0] = memt - ww * (memt * ecol - acol)


def kernel(memory, controls, read_weights):
    b, c, w = memory.shape
    n = controls.shape[-1]
    memt = jnp.swapaxes(memory, 1, 2)
    ctrl3 = controls.reshape(b, 1, n)
    out_t = pl.pallas_call(
        _write_head_kernel,
        grid=(b,),
        in_specs=[
            pl.BlockSpec((1, w, c), lambda i: (i, 0, 0)),
            pl.BlockSpec((1, 1, n), lambda i: (i, 0, 0)),
        ],
        out_specs=pl.BlockSpec((1, w, c), lambda i: (i, 0, 0)),
        out_shape=jax.ShapeDtypeStruct((b, w, c), memory.dtype),
    )(memt, ctrl3)
    return jnp.swapaxes(out_t, 1, 2)
